# baseline (speedup 1.0000x reference)
import jax
import jax.numpy as jnp
from jax.experimental import pallas as pl
from jax.experimental.pallas import tpu as pltpu
from jax.experimental.pallas import tpu_sc as plsc

_N_HEADS = 16
_SPAN = (_N_HEADS - 1) // 2
_N_VALS = 2 * _SPAN + 1


def _sc_profile_kernel(wt_hbm, v2_hbm, w_row, row_buf, *, l, width, nw):
    p0 = l - 8
    nvec = width // 16
    n_tasks = _N_HEADS * 8
    tpw = -(-n_tasks // nw)
    wid = jax.lax.axis_index("s") * 2 + jax.lax.axis_index("c")
    iota = jax.lax.iota(jnp.int32, 16)
    for u in range(tpw):
        t = wid * tpw + u

        @pl.when(t < n_tasks)
        def _task():
            h = t // 8
            s = t % 8
            pltpu.sync_copy(wt_hbm.at[h], w_row)

            def fill(v, carry):
                idx = jnp.clip(iota + v * 16 - s - p0, 0, _N_VALS - 1)
                row_buf[pl.ds(v * 16, 16)] = plsc.load_gather(w_row, [idx])
                return carry

            jax.lax.fori_loop(0, nvec, fill, 0)
            pltpu.sync_copy(row_buf, v2_hbm.at[h, s])


def _emit_kernel(v2_ref, o_ref, *, br, l):
    rb = pl.program_id(1)
    i0 = rb * br
    strip = min(l + 128, 2 * l)

    def body(g, _):
        off = (l - 1) - i0 - 8 * g
        q = off // 128
        m = off - q * 128
        aligned = v2_ref[0, :, pl.ds(q * 128, strip)]
        rolled = pltpu.roll(aligned, strip - m, axis=1)
        o_ref[0, 0, pl.ds(8 * g, 8), :] = rolled[:, :l]
        return 0

    jax.lax.fori_loop(0, br // 8, body, 0, unroll=True)


def kernel(attn, W):
    n_heads = attn.shape[0]
    l = attn.shape[2]
    br = min(2048, l)
    width = 2 * l
    wt = W.T.astype(jnp.float32)
    mesh = plsc.VectorSubcoreMesh(core_axis_name="c", subcore_axis_name="s")
    nw = mesh.num_cores * mesh.num_subcores
    sc_builder = pl.kernel(
        lambda wt_ref, v2_ref, w_row, row_buf: _sc_profile_kernel(
            wt_ref, v2_ref, w_row, row_buf, l=l, width=width, nw=nw),
        out_type=jax.ShapeDtypeStruct((n_heads, 8, width), jnp.float32),
        mesh=mesh,
        scratch_types=[
            pltpu.VMEM((16,), jnp.float32),
            pltpu.VMEM((width,), jnp.float32),
        ],
        compiler_params=pltpu.CompilerParams(needs_layout_passes=False),
    )
    v2 = sc_builder(wt)
    out = pl.pallas_call(
        lambda v2_ref, o_ref: _emit_kernel(v2_ref, o_ref, br=br, l=l),
        grid=(n_heads, l // br),
        in_specs=[pl.BlockSpec((1, 8, width), lambda h, rb: (h, 0, 0))],
        out_specs=pl.BlockSpec((1, 1, br, l), lambda h, rb: (h, 0, rb, 0)),
        out_shape=jax.ShapeDtypeStruct((n_heads, 1, l, l), jnp.float32),
    )(v2)
    return out
